# Initial kernel scaffold; baseline (speedup 1.0000x reference)
import jax
import jax.numpy as jnp
from jax.experimental import pallas as pl


def kernel(x, embedding_weight, offsets):
    raise NotImplementedError("write your pallas kernel here")


# speedup vs baseline: 34.7338x; 34.7338x over previous
import jax
import jax.numpy as jnp
from jax import lax
from jax.experimental import pallas as pl
from jax.experimental.pallas import tpu as pltpu


def _gather_block_kernel(idx_ref, hi_ref, lo_ref, out_ref, *, fields, rows_per_field):
    bsub = idx_ref.shape[0]
    d = out_ref.shape[1] // fields
    for f in range(fields):
        base = f * rows_per_field
        col = idx_ref[:, f : f + 1]
        row_ids = base + lax.broadcasted_iota(
            jnp.int32, (bsub, rows_per_field), 1
        )
        onehot = (col == row_ids).astype(jnp.bfloat16)
        sub_hi = hi_ref[base : base + rows_per_field, :]
        sub_lo = lo_ref[base : base + rows_per_field, :]
        res = jnp.dot(onehot, sub_hi, preferred_element_type=jnp.float32)
        res += jnp.dot(onehot, sub_lo, preferred_element_type=jnp.float32)
        out_ref[:, f * d : (f + 1) * d] = res


def kernel(x, embedding_weight, offsets):
    B, F = x.shape
    V, D = embedding_weight.shape
    rows_per_field = V // F

    g = x.astype(jnp.int32) + offsets.astype(jnp.int32)[None, :]

    hi = embedding_weight.astype(jnp.bfloat16)
    lo = (embedding_weight - hi.astype(jnp.float32)).astype(jnp.bfloat16)

    BSUB = 512
    assert B % BSUB == 0

    out = pl.pallas_call(
        lambda i, h, l, o: _gather_block_kernel(
            i, h, l, o, fields=F, rows_per_field=rows_per_field
        ),
        out_shape=jax.ShapeDtypeStruct((B, F * D), jnp.float32),
        grid=(B // BSUB,),
        in_specs=[
            pl.BlockSpec((BSUB, F), lambda i: (i, 0)),
            pl.BlockSpec((V, D), lambda i: (0, 0)),
            pl.BlockSpec((V, D), lambda i: (0, 0)),
        ],
        out_specs=pl.BlockSpec((BSUB, F * D), lambda i: (i, 0)),
        compiler_params=pltpu.CompilerParams(
            dimension_semantics=("parallel",),
            vmem_limit_bytes=48 * 1024 * 1024,
        ),
    )(g, hi, lo)

    return out.reshape(B, F, D)
